# Initial kernel scaffold; baseline (speedup 1.0000x reference)
import jax
import jax.numpy as jnp
from jax.experimental import pallas as pl


def kernel(x1, x2, x, edge_index1, edge_index2, edge_index, W_gcn, b_gcn, bn1_g, bn1_b, prelu1, W1, b1, bn2_g, bn2_b, prelu2, W2, b2):
    raise NotImplementedError("write your pallas kernel here")


# speedup vs baseline: 15.1801x; 15.1801x over previous
import functools

import jax
import jax.numpy as jnp
from jax import lax
from jax.experimental import pallas as pl
from jax.experimental.pallas import tpu as pltpu
from jax.experimental.pallas import tpu_sc as plsc

N = 10000
E = 320000
D = 128
H = 512
NN = 3 * N
EE = 3 * E
INV_T = 20.0
BN_EPS = 1e-5

_NC = 2
_NS = 16
EB = 80
TILE_E_DEG = EE // (_NC * _NS)
TILE_E_MP = E // (_NC * _NS)
DEG_PAD = 32768
DEG_TILE = DEG_PAD // _NS
NPAD = 10240
RPT = NPAD // _NS
WCH = 128

_mesh = plsc.VectorSubcoreMesh(core_axis_name="c", subcore_axis_name="s")


@functools.partial(
    pl.kernel,
    mesh=_mesh,
    out_type=jax.ShapeDtypeStruct((_NC * DEG_PAD,), jnp.float32),
    scratch_types=[
        pltpu.VMEM((TILE_E_DEG // EB, EB), jnp.int32),
        pltpu.VMEM((1, EB), jnp.float32),
        pltpu.VMEM((DEG_TILE,), jnp.float32),
        pltpu.VMEM_SHARED((DEG_PAD,), jnp.float32),
    ],
)
def _sc_degree(dst_hbm, out_hbm, idx_v, ones_v, stage_v, deg_sh):
    c = lax.axis_index("c")
    s = lax.axis_index("s")
    nb = TILE_E_DEG // EB

    def zero_body(i, _):
        stage_v[pl.ds(i * 16, 16)] = jnp.zeros((16,), jnp.float32)
        return _

    lax.fori_loop(0, DEG_TILE // 16, zero_body, None)
    for j in range(EB // 16):
        ones_v[0, pl.ds(j * 16, 16)] = jnp.ones((16,), jnp.float32)
    pltpu.sync_copy(stage_v, deg_sh.at[pl.ds(s * DEG_TILE, DEG_TILE)])
    plsc.subcore_barrier()

    pltpu.sync_copy(dst_hbm.at[c * _NS + s], idx_v)

    def body(j, _):
        pltpu.sync_copy(ones_v.at[0], deg_sh.at[idx_v.at[j]], add=True)
        return _

    lax.fori_loop(0, nb, body, None)
    plsc.subcore_barrier()
    pltpu.sync_copy(deg_sh.at[pl.ds(s * DEG_TILE, DEG_TILE)], stage_v)
    pltpu.sync_copy(stage_v,
                    out_hbm.at[pl.ds(c * DEG_PAD + s * DEG_TILE, DEG_TILE)])


@functools.partial(
    pl.kernel,
    mesh=_mesh,
    out_type=jax.ShapeDtypeStruct((_NC, 3, NPAD, D), jnp.float32),
    scratch_types=[
        pltpu.VMEM((TILE_E_MP // EB, EB), jnp.int32),
        pltpu.VMEM((TILE_E_MP // EB, EB), jnp.int32),
        pltpu.VMEM((WCH, D), jnp.float32),
        pltpu.SemaphoreType.DMA,
        pltpu.VMEM_SHARED((NPAD, D), jnp.float32),
    ],
)
def _sc_message(g_hbm, src_hbm, dst_hbm, zeros_hbm, out_hbm,
                sidx_v, didx_v, rows_v, sem, acc_sh):
    c = lax.axis_index("c")
    s = lax.axis_index("s")
    nb = TILE_E_MP // EB

    for g in range(3):
        pltpu.sync_copy(zeros_hbm, rows_v)
        for t in range(RPT // WCH):
            pltpu.sync_copy(rows_v,
                            acc_sh.at[pl.ds(s * RPT + t * WCH, WCH)])
        plsc.subcore_barrier()

        tile = g * (_NC * _NS) + c * _NS + s
        pltpu.sync_copy(src_hbm.at[tile], sidx_v)
        pltpu.sync_copy(dst_hbm.at[tile], didx_v)

        def body(j, _):
            pltpu.async_copy(g_hbm.at[sidx_v.at[j]], rows_v.at[pl.ds(0, EB)],
                             sem).wait()
            pltpu.sync_copy(rows_v.at[pl.ds(0, EB)], acc_sh.at[didx_v.at[j]],
                            add=True)
            return _

        lax.fori_loop(0, nb, body, None)
        plsc.subcore_barrier()
        for t in range(RPT // WCH):
            pltpu.sync_copy(acc_sh.at[pl.ds(s * RPT + t * WCH, WCH)],
                            rows_v)
            pltpu.sync_copy(rows_v,
                            out_hbm.at[c, g, pl.ds(s * RPT + t * WCH, WCH)])
        plsc.subcore_barrier()


BM = 1000


def _pre_body(x_ref, w_ref, degp_ref, g_ref):
    deg = degp_ref[0] + degp_ref[1] + 1.0
    dinv = lax.rsqrt(deg)
    g_ref[...] = jnp.dot(x_ref[...], w_ref[...],
                         preferred_element_type=jnp.float32) * dinv


def _tc_pre(x3, w, degp):
    return pl.pallas_call(
        _pre_body,
        grid=(NN // BM,),
        in_specs=[
            pl.BlockSpec((BM, D), lambda i: (i, 0)),
            pl.BlockSpec((D, D), lambda i: (0, 0)),
            pl.BlockSpec((2, BM, 1), lambda i: (0, i, 0)),
        ],
        out_specs=pl.BlockSpec((BM, D), lambda i: (i, 0)),
        out_shape=jax.ShapeDtypeStruct((NN, D), jnp.float32),
    )(x3, w, degp)


def _post_body(mp_ref, g_ref, degp_ref, b_ref, enc_ref, st_ref):
    i = pl.program_id(1)
    deg = degp_ref[0] + degp_ref[1] + 1.0
    dinv = lax.rsqrt(deg)
    val = (mp_ref[0] + mp_ref[1] + g_ref[...]) * dinv + b_ref[...]
    enc_ref[...] = val

    @pl.when(i == 0)
    def _():
        st_ref[...] = jnp.zeros_like(st_ref)

    st_ref[0, 0, :] += jnp.sum(val, axis=0)
    st_ref[0, 1, :] += jnp.sum(val * val, axis=0)


def _tc_post(mp, g, degp, b_gcn):
    return pl.pallas_call(
        _post_body,
        grid=(3, N // BM),
        in_specs=[
            pl.BlockSpec((2, BM, D), lambda k, i: (0, k * 10 + i, 0)),
            pl.BlockSpec((BM, D), lambda k, i: (k * 10 + i, 0)),
            pl.BlockSpec((2, BM, 1), lambda k, i: (0, k * 10 + i, 0)),
            pl.BlockSpec((1, D), lambda k, i: (0, 0)),
        ],
        out_specs=[
            pl.BlockSpec((BM, D), lambda k, i: (k * 10 + i, 0)),
            pl.BlockSpec((1, 2, D), lambda k, i: (k, 0, 0)),
        ],
        out_shape=[
            jax.ShapeDtypeStruct((NN, D), jnp.float32),
            jax.ShapeDtypeStruct((3, 2, D), jnp.float32),
        ],
    )(mp, g, degp, b_gcn)


def _emb_body(enc_ref, bnp_ref, gb_ref, a_ref, out_ref):
    mu = bnp_ref[0, 0, :]
    rstd = bnp_ref[0, 1, :]
    xn = (enc_ref[...] - mu) * rstd * gb_ref[0] + gb_ref[1]
    a = a_ref[0, 0]
    out_ref[...] = jnp.where(xn >= 0, xn, a * xn)


def _tc_emb(enc, bnp, gb, a):
    return pl.pallas_call(
        _emb_body,
        grid=(N // BM,),
        in_specs=[
            pl.BlockSpec((BM, D), lambda i: (20 + i, 0)),
            pl.BlockSpec((1, 2, D), lambda i: (2, 0, 0)),
            pl.BlockSpec((2, D), lambda i: (0, 0)),
            pl.BlockSpec((1, 1), lambda i: (0, 0)),
        ],
        out_specs=pl.BlockSpec((BM, D), lambda i: (i, 0)),
        out_shape=jax.ShapeDtypeStruct((N, D), jnp.float32),
    )(enc, bnp, gb, a)


def _proj1_body(enc_ref, bnp_ref, gb_ref, a_ref, w1_ref, b1_ref, m_ref, st_ref):
    i = pl.program_id(1)
    mu = bnp_ref[0, 0, :]
    rstd = bnp_ref[0, 1, :]
    xn = (enc_ref[...] - mu) * rstd * gb_ref[0] + gb_ref[1]
    a = a_ref[0, 0]
    hn = jnp.where(xn >= 0, xn, a * xn)
    m = jnp.dot(hn, w1_ref[...], preferred_element_type=jnp.float32) + b1_ref[...]
    m_ref[...] = m

    @pl.when(i == 0)
    def _():
        st_ref[...] = jnp.zeros_like(st_ref)

    st_ref[0, 0, :] += jnp.sum(m, axis=0)
    st_ref[0, 1, :] += jnp.sum(m * m, axis=0)


def _tc_proj1(enc, bnp, gb, a, w1, b1):
    return pl.pallas_call(
        _proj1_body,
        grid=(2, N // BM),
        in_specs=[
            pl.BlockSpec((BM, D), lambda k, i: (k * 10 + i, 0)),
            pl.BlockSpec((1, 2, D), lambda k, i: (k, 0, 0)),
            pl.BlockSpec((2, D), lambda k, i: (0, 0)),
            pl.BlockSpec((1, 1), lambda k, i: (0, 0)),
            pl.BlockSpec((D, H), lambda k, i: (0, 0)),
            pl.BlockSpec((1, H), lambda k, i: (0, 0)),
        ],
        out_specs=[
            pl.BlockSpec((BM, H), lambda k, i: (k * 10 + i, 0)),
            pl.BlockSpec((1, 2, H), lambda k, i: (k, 0, 0)),
        ],
        out_shape=[
            jax.ShapeDtypeStruct((2 * N, H), jnp.float32),
            jax.ShapeDtypeStruct((2, 2, H), jnp.float32),
        ],
    )(enc, bnp, gb, a, w1, b1)


def _proj2_body(m_ref, bnp_ref, gb_ref, a_ref, w2_ref, b2_ref, z_ref):
    mu = bnp_ref[0, 0, :]
    rstd = bnp_ref[0, 1, :]
    xn = (m_ref[...] - mu) * rstd * gb_ref[0] + gb_ref[1]
    a = a_ref[0, 0]
    hn = jnp.where(xn >= 0, xn, a * xn)
    h = jnp.dot(hn, w2_ref[...], preferred_element_type=jnp.float32) + b2_ref[...]
    nrm = jnp.maximum(jnp.sqrt(jnp.sum(h * h, axis=1, keepdims=True)), 1e-12)
    z_ref[...] = h / nrm


def _tc_proj2(m, bnp, gb, a, w2, b2):
    return pl.pallas_call(
        _proj2_body,
        grid=(2, N // BM),
        in_specs=[
            pl.BlockSpec((BM, H), lambda k, i: (k * 10 + i, 0)),
            pl.BlockSpec((1, 2, H), lambda k, i: (k, 0, 0)),
            pl.BlockSpec((2, H), lambda k, i: (0, 0)),
            pl.BlockSpec((1, 1), lambda k, i: (0, 0)),
            pl.BlockSpec((H, D), lambda k, i: (0, 0)),
            pl.BlockSpec((1, D), lambda k, i: (0, 0)),
        ],
        out_specs=pl.BlockSpec((BM, D), lambda k, i: (k * 10 + i, 0)),
        out_shape=jax.ShapeDtypeStruct((2 * N, D), jnp.float32),
    )(m, bnp, gb, a, w2, b2)


def _loss_body(z1i_ref, z2i_ref, z1j_ref, z2j_ref, out_ref,
               r1_s, r2_s, bb_s, cc_s, dd_s, r1d_s, r2d_s):
    i = pl.program_id(0)
    j = pl.program_id(1)

    @pl.when((i == 0) & (j == 0))
    def _():
        out_ref[...] = jnp.zeros_like(out_ref)

    @pl.when(j == 0)
    def _():
        r1_s[...] = jnp.zeros_like(r1_s)
        r2_s[...] = jnp.zeros_like(r2_s)
        bb_s[...] = jnp.zeros_like(bb_s)
        cc_s[...] = jnp.zeros_like(cc_s)

    z1i = z1i_ref[...]
    z2i = z2i_ref[...]
    z1j = z1j_ref[...]
    z2j = z2j_ref[...]
    dn = (((1,), (1,)), ((), ()))
    s11 = lax.dot_general(z1i, z1j, dn, preferred_element_type=jnp.float32)
    s22 = lax.dot_general(z2i, z2j, dn, preferred_element_type=jnp.float32)
    s12 = lax.dot_general(z1i, z2j, dn, preferred_element_type=jnp.float32)
    s21 = lax.dot_general(z2i, z1j, dn, preferred_element_type=jnp.float32)
    e11 = jnp.exp(s11 * INV_T)
    e22 = jnp.exp(s22 * INV_T)
    e12 = jnp.exp(s12 * INV_T)
    e21 = jnp.exp(s21 * INV_T)
    r1_s[...] += jnp.sum(e11, axis=1, keepdims=True)
    r2_s[...] += jnp.sum(e22, axis=1, keepdims=True)
    bb_s[...] += jnp.sum(e12, axis=1, keepdims=True)
    cc_s[...] += jnp.sum(e21, axis=1, keepdims=True)

    @pl.when(i == j)
    def _():
        ir = lax.broadcasted_iota(jnp.int32, (BM, BM), 0)
        ic = lax.broadcasted_iota(jnp.int32, (BM, BM), 1)
        dmask = ir == ic
        zero = jnp.zeros_like(s12)
        dd_s[...] = jnp.sum(jnp.where(dmask, s12, zero), axis=1, keepdims=True)
        r1d_s[...] = jnp.sum(jnp.where(dmask, e11, zero), axis=1, keepdims=True)
        r2d_s[...] = jnp.sum(jnp.where(dmask, e22, zero), axis=1, keepdims=True)

    @pl.when(j == (N // BM) - 1)
    def _():
        l1 = jnp.log(r1_s[...] + bb_s[...] - r1d_s[...]) - dd_s[...] * INV_T
        l2 = jnp.log(r2_s[...] + cc_s[...] - r2d_s[...]) - dd_s[...] * INV_T
        out_ref[...] += (0.5 / N) * jnp.sum(l1 + l2).reshape(1, 1)


def _tc_loss(z):
    nb = N // BM
    return pl.pallas_call(
        _loss_body,
        grid=(nb, nb),
        in_specs=[
            pl.BlockSpec((BM, D), lambda i, j: (i, 0)),
            pl.BlockSpec((BM, D), lambda i, j: (nb + i, 0)),
            pl.BlockSpec((BM, D), lambda i, j: (j, 0)),
            pl.BlockSpec((BM, D), lambda i, j: (nb + j, 0)),
        ],
        out_specs=pl.BlockSpec((1, 1), lambda i, j: (0, 0)),
        out_shape=jax.ShapeDtypeStruct((1, 1), jnp.float32),
        scratch_shapes=[pltpu.VMEM((BM, 1), jnp.float32) for _ in range(7)],
    )(z, z, z, z)


def kernel(x1, x2, x, edge_index1, edge_index2, edge_index, W_gcn, b_gcn,
           bn1_g, bn1_b, prelu1, W1, b1, bn2_g, bn2_b, prelu2, W2, b2):
    x3 = jnp.concatenate([x1, x2, x], axis=0)
    off = jnp.int32(N)
    ntile = _NC * _NS
    src3 = jnp.concatenate([edge_index1[0], edge_index2[0] + off,
                            edge_index[0] + 2 * off]
                           ).reshape(3 * ntile, TILE_E_MP // EB, EB)
    dst3 = jnp.concatenate([edge_index1[1], edge_index2[1],
                            edge_index[1]]
                           ).reshape(3 * ntile, TILE_E_MP // EB, EB)
    dstf3 = jnp.concatenate([edge_index1[1], edge_index2[1] + off,
                             edge_index[1] + 2 * off]
                            ).reshape(ntile, TILE_E_DEG // EB, EB)

    degp = _sc_degree(dstf3).reshape(2, DEG_PAD)[:, :NN].reshape(2, NN, 1)
    g = _tc_pre(x3, W_gcn, degp)
    zeros_chunk = jnp.zeros((WCH, D), jnp.float32)
    mp = _sc_message(g, src3, dst3, zeros_chunk)[:, :, :N, :].reshape(
        2, NN, D)

    encpre, st1 = _tc_post(mp, g, degp, b_gcn.reshape(1, D))
    mu1 = st1[:, 0, :] / N
    var1 = st1[:, 1, :] / N - mu1 * mu1
    bnp1 = jnp.stack([mu1, lax.rsqrt(var1 + BN_EPS)], axis=1)
    gb1 = jnp.stack([bn1_g, bn1_b], axis=0)
    a1 = prelu1.reshape(1, 1)

    emb = _tc_emb(encpre, bnp1, gb1, a1)
    m, st2 = _tc_proj1(encpre, bnp1, gb1, a1, W1, b1.reshape(1, H))
    mu2 = st2[:, 0, :] / N
    var2 = st2[:, 1, :] / N - mu2 * mu2
    bnp2 = jnp.stack([mu2, lax.rsqrt(var2 + BN_EPS)], axis=1)
    gb2 = jnp.stack([bn2_g, bn2_b], axis=0)
    z = _tc_proj2(m, bnp2, gb2, prelu2.reshape(1, 1), W2, b2.reshape(1, D))

    loss = _tc_loss(z).reshape(())
    return (emb, loss)


# speedup vs baseline: 16.4575x; 1.0842x over previous
import functools

import jax
import jax.numpy as jnp
from jax import lax
from jax.experimental import pallas as pl
from jax.experimental.pallas import tpu as pltpu
from jax.experimental.pallas import tpu_sc as plsc

N = 10000
E = 320000
D = 128
H = 512
NN = 3 * N
EE = 3 * E
INV_T = 20.0
BN_EPS = 1e-5

_NC = 2
_NS = 16
EB = 80
EBM = 125
TILE_E_DEG = EE // (_NC * _NS)
TILE_E_MP = E // (_NC * _NS)
DEG_PAD = 32768
DEG_TILE = DEG_PAD // _NS
NPAD = 10240
RPT = NPAD // _NS
WCH = 128

_mesh = plsc.VectorSubcoreMesh(core_axis_name="c", subcore_axis_name="s")


@functools.partial(
    pl.kernel,
    mesh=_mesh,
    out_type=jax.ShapeDtypeStruct((_NC * DEG_PAD,), jnp.float32),
    scratch_types=[
        pltpu.VMEM((TILE_E_DEG // EB, EB), jnp.int32),
        pltpu.VMEM((1, EB), jnp.float32),
        pltpu.VMEM((DEG_TILE,), jnp.float32),
        pltpu.VMEM_SHARED((DEG_PAD,), jnp.float32),
    ],
)
def _sc_degree(dst_hbm, out_hbm, idx_v, ones_v, stage_v, deg_sh):
    c = lax.axis_index("c")
    s = lax.axis_index("s")
    nb = TILE_E_DEG // EB

    def zero_body(i, _):
        stage_v[pl.ds(i * 16, 16)] = jnp.zeros((16,), jnp.float32)
        return _

    lax.fori_loop(0, DEG_TILE // 16, zero_body, None)
    for j in range(EB // 16):
        ones_v[0, pl.ds(j * 16, 16)] = jnp.ones((16,), jnp.float32)
    pltpu.sync_copy(stage_v, deg_sh.at[pl.ds(s * DEG_TILE, DEG_TILE)])
    plsc.subcore_barrier()

    pltpu.sync_copy(dst_hbm.at[c * _NS + s], idx_v)

    def body(j, _):
        pltpu.sync_copy(ones_v.at[0], deg_sh.at[idx_v.at[j]], add=True)
        return _

    lax.fori_loop(0, nb, body, None)
    plsc.subcore_barrier()
    pltpu.sync_copy(deg_sh.at[pl.ds(s * DEG_TILE, DEG_TILE)], stage_v)
    pltpu.sync_copy(stage_v,
                    out_hbm.at[pl.ds(c * DEG_PAD + s * DEG_TILE, DEG_TILE)])


@functools.partial(
    pl.kernel,
    mesh=_mesh,
    out_type=jax.ShapeDtypeStruct((_NC, 3, NPAD, D), jnp.float32),
    scratch_types=[
        pltpu.VMEM((TILE_E_MP // EBM, EBM), jnp.int32),
        pltpu.VMEM((TILE_E_MP // EBM, EBM), jnp.int32),
        pltpu.VMEM((WCH, D), jnp.float32),
        pltpu.SemaphoreType.DMA,
        pltpu.VMEM_SHARED((NPAD, D), jnp.float32),
    ],
)
def _sc_message(g_hbm, src_hbm, dst_hbm, zeros_hbm, out_hbm,
                sidx_v, didx_v, rowsa_v, sema, acc_sh):
    c = lax.axis_index("c")
    s = lax.axis_index("s")
    nb = TILE_E_MP // EBM

    for g in range(3):
        pltpu.sync_copy(zeros_hbm, rowsa_v)
        for t in range(RPT // WCH):
            pltpu.sync_copy(rowsa_v,
                            acc_sh.at[pl.ds(s * RPT + t * WCH, WCH)])
        plsc.subcore_barrier()

        tile = g * (_NC * _NS) + c * _NS + s
        pltpu.sync_copy(src_hbm.at[tile], sidx_v)
        pltpu.sync_copy(dst_hbm.at[tile], didx_v)

        def body(j, _):
            pltpu.async_copy(g_hbm.at[sidx_v.at[j]],
                             rowsa_v.at[pl.ds(0, EBM)], sema).wait()
            pltpu.sync_copy(rowsa_v.at[pl.ds(0, EBM)],
                            acc_sh.at[didx_v.at[j]], add=True)
            return _

        lax.fori_loop(0, nb, body, None)
        plsc.subcore_barrier()
        for t in range(RPT // WCH):
            pltpu.sync_copy(acc_sh.at[pl.ds(s * RPT + t * WCH, WCH)],
                            rowsa_v)
            pltpu.sync_copy(rowsa_v,
                            out_hbm.at[c, g, pl.ds(s * RPT + t * WCH, WCH)])
        plsc.subcore_barrier()


BM = 1000


def _pre_body(x_ref, w_ref, degp_ref, g_ref):
    deg = degp_ref[0] + degp_ref[1] + 1.0
    dinv = lax.rsqrt(deg)
    g_ref[...] = jnp.dot(x_ref[...], w_ref[...],
                         preferred_element_type=jnp.float32) * dinv


def _tc_pre(x3, w, degp):
    return pl.pallas_call(
        _pre_body,
        grid=(NN // BM,),
        in_specs=[
            pl.BlockSpec((BM, D), lambda i: (i, 0)),
            pl.BlockSpec((D, D), lambda i: (0, 0)),
            pl.BlockSpec((2, BM, 1), lambda i: (0, i, 0)),
        ],
        out_specs=pl.BlockSpec((BM, D), lambda i: (i, 0)),
        out_shape=jax.ShapeDtypeStruct((NN, D), jnp.float32),
    )(x3, w, degp)


def _post_body(mp_ref, g_ref, degp_ref, b_ref, enc_ref, st_ref):
    i = pl.program_id(1)
    deg = degp_ref[0] + degp_ref[1] + 1.0
    dinv = lax.rsqrt(deg)
    val = (mp_ref[0] + mp_ref[1] + g_ref[...]) * dinv + b_ref[...]
    enc_ref[...] = val

    @pl.when(i == 0)
    def _():
        st_ref[...] = jnp.zeros_like(st_ref)

    st_ref[0, 0, :] += jnp.sum(val, axis=0)
    st_ref[0, 1, :] += jnp.sum(val * val, axis=0)


def _tc_post(mp, g, degp, b_gcn):
    return pl.pallas_call(
        _post_body,
        grid=(3, N // BM),
        in_specs=[
            pl.BlockSpec((2, BM, D), lambda k, i: (0, k * 10 + i, 0)),
            pl.BlockSpec((BM, D), lambda k, i: (k * 10 + i, 0)),
            pl.BlockSpec((2, BM, 1), lambda k, i: (0, k * 10 + i, 0)),
            pl.BlockSpec((1, D), lambda k, i: (0, 0)),
        ],
        out_specs=[
            pl.BlockSpec((BM, D), lambda k, i: (k * 10 + i, 0)),
            pl.BlockSpec((1, 2, D), lambda k, i: (k, 0, 0)),
        ],
        out_shape=[
            jax.ShapeDtypeStruct((NN, D), jnp.float32),
            jax.ShapeDtypeStruct((3, 2, D), jnp.float32),
        ],
    )(mp, g, degp, b_gcn)


def _emb_body(enc_ref, bnp_ref, gb_ref, a_ref, out_ref):
    mu = bnp_ref[0, 0, :]
    rstd = bnp_ref[0, 1, :]
    xn = (enc_ref[...] - mu) * rstd * gb_ref[0] + gb_ref[1]
    a = a_ref[0, 0]
    out_ref[...] = jnp.where(xn >= 0, xn, a * xn)


def _tc_emb(enc, bnp, gb, a):
    return pl.pallas_call(
        _emb_body,
        grid=(N // BM,),
        in_specs=[
            pl.BlockSpec((BM, D), lambda i: (20 + i, 0)),
            pl.BlockSpec((1, 2, D), lambda i: (2, 0, 0)),
            pl.BlockSpec((2, D), lambda i: (0, 0)),
            pl.BlockSpec((1, 1), lambda i: (0, 0)),
        ],
        out_specs=pl.BlockSpec((BM, D), lambda i: (i, 0)),
        out_shape=jax.ShapeDtypeStruct((N, D), jnp.float32),
    )(enc, bnp, gb, a)


def _proj1_body(enc_ref, bnp_ref, gb_ref, a_ref, w1_ref, b1_ref, m_ref, st_ref):
    i = pl.program_id(1)
    mu = bnp_ref[0, 0, :]
    rstd = bnp_ref[0, 1, :]
    xn = (enc_ref[...] - mu) * rstd * gb_ref[0] + gb_ref[1]
    a = a_ref[0, 0]
    hn = jnp.where(xn >= 0, xn, a * xn)
    m = jnp.dot(hn, w1_ref[...], preferred_element_type=jnp.float32) + b1_ref[...]
    m_ref[...] = m

    @pl.when(i == 0)
    def _():
        st_ref[...] = jnp.zeros_like(st_ref)

    st_ref[0, 0, :] += jnp.sum(m, axis=0)
    st_ref[0, 1, :] += jnp.sum(m * m, axis=0)


def _tc_proj1(enc, bnp, gb, a, w1, b1):
    return pl.pallas_call(
        _proj1_body,
        grid=(2, N // BM),
        in_specs=[
            pl.BlockSpec((BM, D), lambda k, i: (k * 10 + i, 0)),
            pl.BlockSpec((1, 2, D), lambda k, i: (k, 0, 0)),
            pl.BlockSpec((2, D), lambda k, i: (0, 0)),
            pl.BlockSpec((1, 1), lambda k, i: (0, 0)),
            pl.BlockSpec((D, H), lambda k, i: (0, 0)),
            pl.BlockSpec((1, H), lambda k, i: (0, 0)),
        ],
        out_specs=[
            pl.BlockSpec((BM, H), lambda k, i: (k * 10 + i, 0)),
            pl.BlockSpec((1, 2, H), lambda k, i: (k, 0, 0)),
        ],
        out_shape=[
            jax.ShapeDtypeStruct((2 * N, H), jnp.float32),
            jax.ShapeDtypeStruct((2, 2, H), jnp.float32),
        ],
    )(enc, bnp, gb, a, w1, b1)


def _proj2_body(m_ref, bnp_ref, gb_ref, a_ref, w2_ref, b2_ref, z_ref):
    mu = bnp_ref[0, 0, :]
    rstd = bnp_ref[0, 1, :]
    xn = (m_ref[...] - mu) * rstd * gb_ref[0] + gb_ref[1]
    a = a_ref[0, 0]
    hn = jnp.where(xn >= 0, xn, a * xn)
    h = jnp.dot(hn, w2_ref[...], preferred_element_type=jnp.float32) + b2_ref[...]
    nrm = jnp.maximum(jnp.sqrt(jnp.sum(h * h, axis=1, keepdims=True)), 1e-12)
    z_ref[...] = h / nrm


def _tc_proj2(m, bnp, gb, a, w2, b2):
    return pl.pallas_call(
        _proj2_body,
        grid=(2, N // BM),
        in_specs=[
            pl.BlockSpec((BM, H), lambda k, i: (k * 10 + i, 0)),
            pl.BlockSpec((1, 2, H), lambda k, i: (k, 0, 0)),
            pl.BlockSpec((2, H), lambda k, i: (0, 0)),
            pl.BlockSpec((1, 1), lambda k, i: (0, 0)),
            pl.BlockSpec((H, D), lambda k, i: (0, 0)),
            pl.BlockSpec((1, D), lambda k, i: (0, 0)),
        ],
        out_specs=pl.BlockSpec((BM, D), lambda k, i: (k * 10 + i, 0)),
        out_shape=jax.ShapeDtypeStruct((2 * N, D), jnp.float32),
    )(m, bnp, gb, a, w2, b2)


def _loss_body(z1i_ref, z2i_ref, z1j_ref, z2j_ref, out_ref,
               r1_s, r2_s, bb_s, cc_s, dd_s, r1d_s, r2d_s):
    i = pl.program_id(0)
    j = pl.program_id(1)

    @pl.when((i == 0) & (j == 0))
    def _():
        out_ref[...] = jnp.zeros_like(out_ref)

    @pl.when(j == 0)
    def _():
        r1_s[...] = jnp.zeros_like(r1_s)
        r2_s[...] = jnp.zeros_like(r2_s)
        bb_s[...] = jnp.zeros_like(bb_s)
        cc_s[...] = jnp.zeros_like(cc_s)

    z1i = z1i_ref[...]
    z2i = z2i_ref[...]
    z1j = z1j_ref[...]
    z2j = z2j_ref[...]
    dn = (((1,), (1,)), ((), ()))
    s11 = lax.dot_general(z1i, z1j, dn, preferred_element_type=jnp.float32)
    s22 = lax.dot_general(z2i, z2j, dn, preferred_element_type=jnp.float32)
    s12 = lax.dot_general(z1i, z2j, dn, preferred_element_type=jnp.float32)
    s21 = lax.dot_general(z2i, z1j, dn, preferred_element_type=jnp.float32)
    e11 = jnp.exp(s11 * INV_T)
    e22 = jnp.exp(s22 * INV_T)
    e12 = jnp.exp(s12 * INV_T)
    e21 = jnp.exp(s21 * INV_T)
    r1_s[...] += jnp.sum(e11, axis=1, keepdims=True)
    r2_s[...] += jnp.sum(e22, axis=1, keepdims=True)
    bb_s[...] += jnp.sum(e12, axis=1, keepdims=True)
    cc_s[...] += jnp.sum(e21, axis=1, keepdims=True)

    @pl.when(i == j)
    def _():
        ir = lax.broadcasted_iota(jnp.int32, (BM, BM), 0)
        ic = lax.broadcasted_iota(jnp.int32, (BM, BM), 1)
        dmask = ir == ic
        zero = jnp.zeros_like(s12)
        dd_s[...] = jnp.sum(jnp.where(dmask, s12, zero), axis=1, keepdims=True)
        r1d_s[...] = jnp.sum(jnp.where(dmask, e11, zero), axis=1, keepdims=True)
        r2d_s[...] = jnp.sum(jnp.where(dmask, e22, zero), axis=1, keepdims=True)

    @pl.when(j == (N // BM) - 1)
    def _():
        l1 = jnp.log(r1_s[...] + bb_s[...] - r1d_s[...]) - dd_s[...] * INV_T
        l2 = jnp.log(r2_s[...] + cc_s[...] - r2d_s[...]) - dd_s[...] * INV_T
        out_ref[...] += (0.5 / N) * jnp.sum(l1 + l2).reshape(1, 1)


def _tc_loss(z):
    nb = N // BM
    return pl.pallas_call(
        _loss_body,
        grid=(nb, nb),
        in_specs=[
            pl.BlockSpec((BM, D), lambda i, j: (i, 0)),
            pl.BlockSpec((BM, D), lambda i, j: (nb + i, 0)),
            pl.BlockSpec((BM, D), lambda i, j: (j, 0)),
            pl.BlockSpec((BM, D), lambda i, j: (nb + j, 0)),
        ],
        out_specs=pl.BlockSpec((1, 1), lambda i, j: (0, 0)),
        out_shape=jax.ShapeDtypeStruct((1, 1), jnp.float32),
        scratch_shapes=[pltpu.VMEM((BM, 1), jnp.float32) for _ in range(7)],
    )(z, z, z, z)


def kernel(x1, x2, x, edge_index1, edge_index2, edge_index, W_gcn, b_gcn,
           bn1_g, bn1_b, prelu1, W1, b1, bn2_g, bn2_b, prelu2, W2, b2):
    x3 = jnp.concatenate([x1, x2, x], axis=0)
    off = jnp.int32(N)
    ntile = _NC * _NS
    src3 = jnp.concatenate([edge_index1[0], edge_index2[0] + off,
                            edge_index[0] + 2 * off]
                           ).reshape(3 * ntile, TILE_E_MP // EBM, EBM)
    dst3 = jnp.concatenate([edge_index1[1], edge_index2[1],
                            edge_index[1]]
                           ).reshape(3 * ntile, TILE_E_MP // EBM, EBM)
    dstf3 = jnp.concatenate([edge_index1[1], edge_index2[1] + off,
                             edge_index[1] + 2 * off]
                            ).reshape(ntile, TILE_E_DEG // EB, EB)

    degp = _sc_degree(dstf3).reshape(2, DEG_PAD)[:, :NN].reshape(2, NN, 1)
    g = _tc_pre(x3, W_gcn, degp)
    zeros_chunk = jnp.zeros((WCH, D), jnp.float32)
    mp = _sc_message(g, src3, dst3, zeros_chunk)[:, :, :N, :].reshape(
        2, NN, D)

    encpre, st1 = _tc_post(mp, g, degp, b_gcn.reshape(1, D))
    mu1 = st1[:, 0, :] / N
    var1 = st1[:, 1, :] / N - mu1 * mu1
    bnp1 = jnp.stack([mu1, lax.rsqrt(var1 + BN_EPS)], axis=1)
    gb1 = jnp.stack([bn1_g, bn1_b], axis=0)
    a1 = prelu1.reshape(1, 1)

    emb = _tc_emb(encpre, bnp1, gb1, a1)
    m, st2 = _tc_proj1(encpre, bnp1, gb1, a1, W1, b1.reshape(1, H))
    mu2 = st2[:, 0, :] / N
    var2 = st2[:, 1, :] / N - mu2 * mu2
    bnp2 = jnp.stack([mu2, lax.rsqrt(var2 + BN_EPS)], axis=1)
    gb2 = jnp.stack([bn2_g, bn2_b], axis=0)
    z = _tc_proj2(m, bnp2, gb2, prelu2.reshape(1, 1), W2, b2.reshape(1, D))

    loss = _tc_loss(z).reshape(())
    return (emb, loss)


# speedup vs baseline: 17.4263x; 1.0589x over previous
import functools

import jax
import jax.numpy as jnp
from jax import lax
from jax.experimental import pallas as pl
from jax.experimental.pallas import tpu as pltpu
from jax.experimental.pallas import tpu_sc as plsc

N = 10000
E = 320000
D = 128
H = 512
NN = 3 * N
EE = 3 * E
INV_T = 20.0
BN_EPS = 1e-5

_NC = 2
_NS = 16
EB = 80
EBM = 125
TILE_E_DEG = EE // (_NC * _NS)
TILE_E_MP = E // (_NC * _NS)
DEG_PAD = 32768
DEG_TILE = DEG_PAD // _NS
NPAD = 10240
RPT = NPAD // _NS
WCH = 128

_mesh = plsc.VectorSubcoreMesh(core_axis_name="c", subcore_axis_name="s")


@functools.partial(
    pl.kernel,
    mesh=_mesh,
    out_type=jax.ShapeDtypeStruct((_NC * DEG_PAD,), jnp.float32),
    scratch_types=[
        pltpu.VMEM((TILE_E_DEG // EB, EB), jnp.int32),
        pltpu.VMEM((1, EB), jnp.float32),
        pltpu.VMEM((DEG_TILE,), jnp.float32),
        pltpu.VMEM_SHARED((DEG_PAD,), jnp.float32),
    ],
)
def _sc_degree(dst_hbm, out_hbm, idx_v, ones_v, stage_v, deg_sh):
    c = lax.axis_index("c")
    s = lax.axis_index("s")
    nb = TILE_E_DEG // EB

    def zero_body(i, _):
        stage_v[pl.ds(i * 16, 16)] = jnp.zeros((16,), jnp.float32)
        return _

    lax.fori_loop(0, DEG_TILE // 16, zero_body, None)
    for j in range(EB // 16):
        ones_v[0, pl.ds(j * 16, 16)] = jnp.ones((16,), jnp.float32)
    pltpu.sync_copy(stage_v, deg_sh.at[pl.ds(s * DEG_TILE, DEG_TILE)])
    plsc.subcore_barrier()

    pltpu.sync_copy(dst_hbm.at[c * _NS + s], idx_v)

    def body(j, _):
        pltpu.sync_copy(ones_v.at[0], deg_sh.at[idx_v.at[j]], add=True)
        return _

    lax.fori_loop(0, nb, body, None)
    plsc.subcore_barrier()
    pltpu.sync_copy(deg_sh.at[pl.ds(s * DEG_TILE, DEG_TILE)], stage_v)
    pltpu.sync_copy(stage_v,
                    out_hbm.at[pl.ds(c * DEG_PAD + s * DEG_TILE, DEG_TILE)])


CHB = 16


@functools.partial(
    pl.kernel,
    mesh=_mesh,
    out_type=jax.ShapeDtypeStruct((_NC, 3, NPAD, D), jnp.float32),
    scratch_types=[
        pltpu.VMEM((CHB, EBM), jnp.int32),
        pltpu.VMEM((CHB, EBM), jnp.int32),
        pltpu.VMEM((WCH, D), jnp.float32),
        pltpu.VMEM((EBM, D), jnp.float32),
        pltpu.SemaphoreType.DMA,
        pltpu.SemaphoreType.DMA,
        pltpu.VMEM_SHARED((NPAD, D), jnp.float32),
    ],
)
def _sc_message(g_hbm, src_hbm, dst_hbm, zeros_hbm, out_hbm,
                sidx_v, didx_v, rowsa_v, rowsb_v, sema, semb, acc_sh):
    c = lax.axis_index("c")
    s = lax.axis_index("s")
    nb = TILE_E_MP // EBM

    for g in range(3):
        pltpu.sync_copy(zeros_hbm, rowsa_v)
        for t in range(RPT // WCH):
            pltpu.sync_copy(rowsa_v,
                            acc_sh.at[pl.ds(s * RPT + t * WCH, WCH)])
        plsc.subcore_barrier()

        tile = g * (_NC * _NS) + c * _NS + s

        def body(j2, _):
            j = 2 * j2
            cpa = pltpu.async_copy(g_hbm.at[sidx_v.at[j]],
                                   rowsa_v.at[pl.ds(0, EBM)], sema)
            cpb = pltpu.async_copy(g_hbm.at[sidx_v.at[j + 1]],
                                   rowsb_v, semb)
            cpa.wait()
            pltpu.sync_copy(rowsa_v.at[pl.ds(0, EBM)],
                            acc_sh.at[didx_v.at[j]], add=True)
            cpb.wait()
            pltpu.sync_copy(rowsb_v,
                            acc_sh.at[didx_v.at[j + 1]], add=True)
            return _

        for k in range(nb // CHB):
            pltpu.sync_copy(src_hbm.at[tile, pl.ds(k * CHB, CHB)], sidx_v)
            pltpu.sync_copy(dst_hbm.at[tile, pl.ds(k * CHB, CHB)], didx_v)
            lax.fori_loop(0, CHB // 2, body, None)
        plsc.subcore_barrier()
        for t in range(RPT // WCH):
            pltpu.sync_copy(acc_sh.at[pl.ds(s * RPT + t * WCH, WCH)],
                            rowsa_v)
            pltpu.sync_copy(rowsa_v,
                            out_hbm.at[c, g, pl.ds(s * RPT + t * WCH, WCH)])
        plsc.subcore_barrier()


BM = 1000


def _pre_body(x_ref, w_ref, degp_ref, g_ref):
    deg = degp_ref[0] + degp_ref[1] + 1.0
    dinv = lax.rsqrt(deg)
    g_ref[...] = jnp.dot(x_ref[...], w_ref[...],
                         preferred_element_type=jnp.float32) * dinv


def _tc_pre(x3, w, degp):
    return pl.pallas_call(
        _pre_body,
        grid=(NN // BM,),
        in_specs=[
            pl.BlockSpec((BM, D), lambda i: (i, 0)),
            pl.BlockSpec((D, D), lambda i: (0, 0)),
            pl.BlockSpec((2, BM, 1), lambda i: (0, i, 0)),
        ],
        out_specs=pl.BlockSpec((BM, D), lambda i: (i, 0)),
        out_shape=jax.ShapeDtypeStruct((NN, D), jnp.float32),
    )(x3, w, degp)


def _post_body(mp_ref, g_ref, degp_ref, b_ref, enc_ref, st_ref):
    i = pl.program_id(1)
    deg = degp_ref[0] + degp_ref[1] + 1.0
    dinv = lax.rsqrt(deg)
    val = (mp_ref[0] + mp_ref[1] + g_ref[...]) * dinv + b_ref[...]
    enc_ref[...] = val

    @pl.when(i == 0)
    def _():
        st_ref[...] = jnp.zeros_like(st_ref)

    st_ref[0, 0, :] += jnp.sum(val, axis=0)
    st_ref[0, 1, :] += jnp.sum(val * val, axis=0)


def _tc_post(mp, g, degp, b_gcn):
    return pl.pallas_call(
        _post_body,
        grid=(3, N // BM),
        in_specs=[
            pl.BlockSpec((2, BM, D), lambda k, i: (0, k * 10 + i, 0)),
            pl.BlockSpec((BM, D), lambda k, i: (k * 10 + i, 0)),
            pl.BlockSpec((2, BM, 1), lambda k, i: (0, k * 10 + i, 0)),
            pl.BlockSpec((1, D), lambda k, i: (0, 0)),
        ],
        out_specs=[
            pl.BlockSpec((BM, D), lambda k, i: (k * 10 + i, 0)),
            pl.BlockSpec((1, 2, D), lambda k, i: (k, 0, 0)),
        ],
        out_shape=[
            jax.ShapeDtypeStruct((NN, D), jnp.float32),
            jax.ShapeDtypeStruct((3, 2, D), jnp.float32),
        ],
    )(mp, g, degp, b_gcn)


def _emb_body(enc_ref, bnp_ref, gb_ref, a_ref, out_ref):
    mu = bnp_ref[0, 0, :]
    rstd = bnp_ref[0, 1, :]
    xn = (enc_ref[...] - mu) * rstd * gb_ref[0] + gb_ref[1]
    a = a_ref[0, 0]
    out_ref[...] = jnp.where(xn >= 0, xn, a * xn)


def _tc_emb(enc, bnp, gb, a):
    return pl.pallas_call(
        _emb_body,
        grid=(N // BM,),
        in_specs=[
            pl.BlockSpec((BM, D), lambda i: (20 + i, 0)),
            pl.BlockSpec((1, 2, D), lambda i: (2, 0, 0)),
            pl.BlockSpec((2, D), lambda i: (0, 0)),
            pl.BlockSpec((1, 1), lambda i: (0, 0)),
        ],
        out_specs=pl.BlockSpec((BM, D), lambda i: (i, 0)),
        out_shape=jax.ShapeDtypeStruct((N, D), jnp.float32),
    )(enc, bnp, gb, a)


def _proj1_body(enc_ref, bnp_ref, gb_ref, a_ref, w1_ref, b1_ref, m_ref, st_ref):
    i = pl.program_id(1)
    mu = bnp_ref[0, 0, :]
    rstd = bnp_ref[0, 1, :]
    xn = (enc_ref[...] - mu) * rstd * gb_ref[0] + gb_ref[1]
    a = a_ref[0, 0]
    hn = jnp.where(xn >= 0, xn, a * xn)
    m = jnp.dot(hn, w1_ref[...], preferred_element_type=jnp.float32) + b1_ref[...]
    m_ref[...] = m

    @pl.when(i == 0)
    def _():
        st_ref[...] = jnp.zeros_like(st_ref)

    st_ref[0, 0, :] += jnp.sum(m, axis=0)
    st_ref[0, 1, :] += jnp.sum(m * m, axis=0)


def _tc_proj1(enc, bnp, gb, a, w1, b1):
    return pl.pallas_call(
        _proj1_body,
        grid=(2, N // BM),
        in_specs=[
            pl.BlockSpec((BM, D), lambda k, i: (k * 10 + i, 0)),
            pl.BlockSpec((1, 2, D), lambda k, i: (k, 0, 0)),
            pl.BlockSpec((2, D), lambda k, i: (0, 0)),
            pl.BlockSpec((1, 1), lambda k, i: (0, 0)),
            pl.BlockSpec((D, H), lambda k, i: (0, 0)),
            pl.BlockSpec((1, H), lambda k, i: (0, 0)),
        ],
        out_specs=[
            pl.BlockSpec((BM, H), lambda k, i: (k * 10 + i, 0)),
            pl.BlockSpec((1, 2, H), lambda k, i: (k, 0, 0)),
        ],
        out_shape=[
            jax.ShapeDtypeStruct((2 * N, H), jnp.float32),
            jax.ShapeDtypeStruct((2, 2, H), jnp.float32),
        ],
    )(enc, bnp, gb, a, w1, b1)


def _proj2_body(m_ref, bnp_ref, gb_ref, a_ref, w2_ref, b2_ref, z_ref):
    mu = bnp_ref[0, 0, :]
    rstd = bnp_ref[0, 1, :]
    xn = (m_ref[...] - mu) * rstd * gb_ref[0] + gb_ref[1]
    a = a_ref[0, 0]
    hn = jnp.where(xn >= 0, xn, a * xn)
    h = jnp.dot(hn, w2_ref[...], preferred_element_type=jnp.float32) + b2_ref[...]
    nrm = jnp.maximum(jnp.sqrt(jnp.sum(h * h, axis=1, keepdims=True)), 1e-12)
    z_ref[...] = h / nrm


def _tc_proj2(m, bnp, gb, a, w2, b2):
    return pl.pallas_call(
        _proj2_body,
        grid=(2, N // BM),
        in_specs=[
            pl.BlockSpec((BM, H), lambda k, i: (k * 10 + i, 0)),
            pl.BlockSpec((1, 2, H), lambda k, i: (k, 0, 0)),
            pl.BlockSpec((2, H), lambda k, i: (0, 0)),
            pl.BlockSpec((1, 1), lambda k, i: (0, 0)),
            pl.BlockSpec((H, D), lambda k, i: (0, 0)),
            pl.BlockSpec((1, D), lambda k, i: (0, 0)),
        ],
        out_specs=pl.BlockSpec((BM, D), lambda k, i: (k * 10 + i, 0)),
        out_shape=jax.ShapeDtypeStruct((2 * N, D), jnp.float32),
    )(m, bnp, gb, a, w2, b2)


def _loss_body(z1i_ref, z2i_ref, z1j_ref, z2j_ref, out_ref,
               r1_s, r2_s, bb_s, cc_s, dd_s, r1d_s, r2d_s):
    i = pl.program_id(0)
    j = pl.program_id(1)

    @pl.when((i == 0) & (j == 0))
    def _():
        out_ref[...] = jnp.zeros_like(out_ref)

    @pl.when(j == 0)
    def _():
        r1_s[...] = jnp.zeros_like(r1_s)
        r2_s[...] = jnp.zeros_like(r2_s)
        bb_s[...] = jnp.zeros_like(bb_s)
        cc_s[...] = jnp.zeros_like(cc_s)

    z1i = z1i_ref[...]
    z2i = z2i_ref[...]
    z1j = z1j_ref[...]
    z2j = z2j_ref[...]
    dn = (((1,), (1,)), ((), ()))
    s11 = lax.dot_general(z1i, z1j, dn, preferred_element_type=jnp.float32)
    s22 = lax.dot_general(z2i, z2j, dn, preferred_element_type=jnp.float32)
    s12 = lax.dot_general(z1i, z2j, dn, preferred_element_type=jnp.float32)
    s21 = lax.dot_general(z2i, z1j, dn, preferred_element_type=jnp.float32)
    e11 = jnp.exp(s11 * INV_T)
    e22 = jnp.exp(s22 * INV_T)
    e12 = jnp.exp(s12 * INV_T)
    e21 = jnp.exp(s21 * INV_T)
    r1_s[...] += jnp.sum(e11, axis=1, keepdims=True)
    r2_s[...] += jnp.sum(e22, axis=1, keepdims=True)
    bb_s[...] += jnp.sum(e12, axis=1, keepdims=True)
    cc_s[...] += jnp.sum(e21, axis=1, keepdims=True)

    @pl.when(i == j)
    def _():
        ir = lax.broadcasted_iota(jnp.int32, (BM, BM), 0)
        ic = lax.broadcasted_iota(jnp.int32, (BM, BM), 1)
        dmask = ir == ic
        zero = jnp.zeros_like(s12)
        dd_s[...] = jnp.sum(jnp.where(dmask, s12, zero), axis=1, keepdims=True)
        r1d_s[...] = jnp.sum(jnp.where(dmask, e11, zero), axis=1, keepdims=True)
        r2d_s[...] = jnp.sum(jnp.where(dmask, e22, zero), axis=1, keepdims=True)

    @pl.when(j == (N // BM) - 1)
    def _():
        l1 = jnp.log(r1_s[...] + bb_s[...] - r1d_s[...]) - dd_s[...] * INV_T
        l2 = jnp.log(r2_s[...] + cc_s[...] - r2d_s[...]) - dd_s[...] * INV_T
        out_ref[...] += (0.5 / N) * jnp.sum(l1 + l2).reshape(1, 1)


def _tc_loss(z):
    nb = N // BM
    return pl.pallas_call(
        _loss_body,
        grid=(nb, nb),
        in_specs=[
            pl.BlockSpec((BM, D), lambda i, j: (i, 0)),
            pl.BlockSpec((BM, D), lambda i, j: (nb + i, 0)),
            pl.BlockSpec((BM, D), lambda i, j: (j, 0)),
            pl.BlockSpec((BM, D), lambda i, j: (nb + j, 0)),
        ],
        out_specs=pl.BlockSpec((1, 1), lambda i, j: (0, 0)),
        out_shape=jax.ShapeDtypeStruct((1, 1), jnp.float32),
        scratch_shapes=[pltpu.VMEM((BM, 1), jnp.float32) for _ in range(7)],
    )(z, z, z, z)


def kernel(x1, x2, x, edge_index1, edge_index2, edge_index, W_gcn, b_gcn,
           bn1_g, bn1_b, prelu1, W1, b1, bn2_g, bn2_b, prelu2, W2, b2):
    x3 = jnp.concatenate([x1, x2, x], axis=0)
    off = jnp.int32(N)
    ntile = _NC * _NS
    src3 = jnp.concatenate([edge_index1[0], edge_index2[0] + off,
                            edge_index[0] + 2 * off]
                           ).reshape(3 * ntile, TILE_E_MP // EBM, EBM)
    dst3 = jnp.concatenate([edge_index1[1], edge_index2[1],
                            edge_index[1]]
                           ).reshape(3 * ntile, TILE_E_MP // EBM, EBM)
    dstf3 = jnp.concatenate([edge_index1[1], edge_index2[1] + off,
                             edge_index[1] + 2 * off]
                            ).reshape(ntile, TILE_E_DEG // EB, EB)

    degp = _sc_degree(dstf3).reshape(2, DEG_PAD)[:, :NN].reshape(2, NN, 1)
    g = _tc_pre(x3, W_gcn, degp)
    zeros_chunk = jnp.zeros((WCH, D), jnp.float32)
    mp = _sc_message(g, src3, dst3, zeros_chunk)[:, :, :N, :].reshape(
        2, NN, D)

    encpre, st1 = _tc_post(mp, g, degp, b_gcn.reshape(1, D))
    mu1 = st1[:, 0, :] / N
    var1 = st1[:, 1, :] / N - mu1 * mu1
    bnp1 = jnp.stack([mu1, lax.rsqrt(var1 + BN_EPS)], axis=1)
    gb1 = jnp.stack([bn1_g, bn1_b], axis=0)
    a1 = prelu1.reshape(1, 1)

    emb = _tc_emb(encpre, bnp1, gb1, a1)
    m, st2 = _tc_proj1(encpre, bnp1, gb1, a1, W1, b1.reshape(1, H))
    mu2 = st2[:, 0, :] / N
    var2 = st2[:, 1, :] / N - mu2 * mu2
    bnp2 = jnp.stack([mu2, lax.rsqrt(var2 + BN_EPS)], axis=1)
    gb2 = jnp.stack([bn2_g, bn2_b], axis=0)
    z = _tc_proj2(m, bnp2, gb2, prelu2.reshape(1, 1), W2, b2.reshape(1, D))

    loss = _tc_loss(z).reshape(())
    return (emb, loss)


# speedup vs baseline: 17.5592x; 1.0076x over previous
import functools

import jax
import jax.numpy as jnp
from jax import lax
from jax.experimental import pallas as pl
from jax.experimental.pallas import tpu as pltpu
from jax.experimental.pallas import tpu_sc as plsc

N = 10000
E = 320000
D = 128
H = 512
NN = 3 * N
EE = 3 * E
INV_T = 20.0
BN_EPS = 1e-5

_NC = 2
_NS = 16
EB = 80
EBM = 125
TILE_E_DEG = EE // (_NC * _NS)
TILE_E_MP = E // (_NC * _NS)
DEG_PAD = 32768
DEG_TILE = DEG_PAD // _NS
NPAD = 10240
RPT = NPAD // _NS
WCH = 128

_mesh = plsc.VectorSubcoreMesh(core_axis_name="c", subcore_axis_name="s")


@functools.partial(
    pl.kernel,
    mesh=_mesh,
    out_type=jax.ShapeDtypeStruct((_NC * DEG_PAD,), jnp.float32),
    scratch_types=[
        pltpu.VMEM((TILE_E_DEG // EB, EB), jnp.int32),
        pltpu.VMEM((1, EB), jnp.float32),
        pltpu.VMEM((DEG_TILE,), jnp.float32),
        pltpu.VMEM_SHARED((DEG_PAD,), jnp.float32),
    ],
)
def _sc_degree(dst_hbm, out_hbm, idx_v, ones_v, stage_v, deg_sh):
    c = lax.axis_index("c")
    s = lax.axis_index("s")
    nb = TILE_E_DEG // EB

    def zero_body(i, _):
        stage_v[pl.ds(i * 16, 16)] = jnp.zeros((16,), jnp.float32)
        return _

    lax.fori_loop(0, DEG_TILE // 16, zero_body, None)
    for j in range(EB // 16):
        ones_v[0, pl.ds(j * 16, 16)] = jnp.ones((16,), jnp.float32)
    pltpu.sync_copy(stage_v, deg_sh.at[pl.ds(s * DEG_TILE, DEG_TILE)])
    plsc.subcore_barrier()

    pltpu.sync_copy(dst_hbm.at[c * _NS + s], idx_v)

    def body(j, _):
        pltpu.sync_copy(ones_v.at[0], deg_sh.at[idx_v.at[j]], add=True)
        return _

    lax.fori_loop(0, nb, body, None)
    plsc.subcore_barrier()
    pltpu.sync_copy(deg_sh.at[pl.ds(s * DEG_TILE, DEG_TILE)], stage_v)
    pltpu.sync_copy(stage_v,
                    out_hbm.at[pl.ds(c * DEG_PAD + s * DEG_TILE, DEG_TILE)])


CHB = 16


def _make_sc_message(ng):
  @functools.partial(
      pl.kernel,
      mesh=_mesh,
      out_type=jax.ShapeDtypeStruct((_NC, ng, NPAD, D), jnp.float32),
      scratch_types=[
          pltpu.VMEM((CHB, EBM), jnp.int32),
          pltpu.VMEM((CHB, EBM), jnp.int32),
          pltpu.VMEM((WCH, D), jnp.float32),
          pltpu.VMEM((EBM, D), jnp.float32),
          pltpu.SemaphoreType.DMA,
          pltpu.SemaphoreType.DMA,
          pltpu.VMEM_SHARED((NPAD, D), jnp.float32),
      ],
  )
  def _sc_message(g_hbm, src_hbm, dst_hbm, zeros_hbm, out_hbm,
                  sidx_v, didx_v, rowsa_v, rowsb_v, sema, semb, acc_sh):
    c = lax.axis_index("c")
    s = lax.axis_index("s")
    nb = TILE_E_MP // EBM

    for g in range(ng):
        pltpu.sync_copy(zeros_hbm, rowsa_v)
        for t in range(RPT // WCH):
            pltpu.sync_copy(rowsa_v,
                            acc_sh.at[pl.ds(s * RPT + t * WCH, WCH)])
        plsc.subcore_barrier()

        tile = g * (_NC * _NS) + c * _NS + s

        def body(j2, _):
            j = 2 * j2
            cpa = pltpu.async_copy(g_hbm.at[sidx_v.at[j]],
                                   rowsa_v.at[pl.ds(0, EBM)], sema)
            cpb = pltpu.async_copy(g_hbm.at[sidx_v.at[j + 1]],
                                   rowsb_v, semb)
            cpa.wait()
            pltpu.sync_copy(rowsa_v.at[pl.ds(0, EBM)],
                            acc_sh.at[didx_v.at[j]], add=True)
            cpb.wait()
            pltpu.sync_copy(rowsb_v,
                            acc_sh.at[didx_v.at[j + 1]], add=True)
            return _

        for k in range(nb // CHB):
            pltpu.sync_copy(src_hbm.at[tile, pl.ds(k * CHB, CHB)], sidx_v)
            pltpu.sync_copy(dst_hbm.at[tile, pl.ds(k * CHB, CHB)], didx_v)
            lax.fori_loop(0, CHB // 2, body, None)
        plsc.subcore_barrier()
        for t in range(RPT // WCH):
            pltpu.sync_copy(acc_sh.at[pl.ds(s * RPT + t * WCH, WCH)],
                            rowsa_v)
            pltpu.sync_copy(rowsa_v,
                            out_hbm.at[c, g, pl.ds(s * RPT + t * WCH, WCH)])
        plsc.subcore_barrier()

  return _sc_message


_sc_message2 = _make_sc_message(2)
_sc_message1 = _make_sc_message(1)


BM = 1000


def _pre_body(x_ref, w_ref, degp_ref, g_ref):
    deg = degp_ref[0] + degp_ref[1] + 1.0
    dinv = lax.rsqrt(deg)
    g_ref[...] = jnp.dot(x_ref[...], w_ref[...],
                         preferred_element_type=jnp.float32) * dinv


def _tc_pre(x3, w, degp):
    return pl.pallas_call(
        _pre_body,
        grid=(NN // BM,),
        in_specs=[
            pl.BlockSpec((BM, D), lambda i: (i, 0)),
            pl.BlockSpec((D, D), lambda i: (0, 0)),
            pl.BlockSpec((2, BM, 1), lambda i: (0, i, 0)),
        ],
        out_specs=pl.BlockSpec((BM, D), lambda i: (i, 0)),
        out_shape=jax.ShapeDtypeStruct((NN, D), jnp.float32),
    )(x3, w, degp)


def _post_body(mp_ref, g_ref, degp_ref, b_ref, enc_ref, st_ref):
    i = pl.program_id(1)
    deg = degp_ref[0] + degp_ref[1] + 1.0
    dinv = lax.rsqrt(deg)
    val = (mp_ref[0] + mp_ref[1] + g_ref[...]) * dinv + b_ref[...]
    enc_ref[...] = val

    @pl.when(i == 0)
    def _():
        st_ref[...] = jnp.zeros_like(st_ref)

    st_ref[0, 0, :] += jnp.sum(val, axis=0)
    st_ref[0, 1, :] += jnp.sum(val * val, axis=0)


def _tc_post(mp, g, degp, b_gcn, ng):
    return pl.pallas_call(
        _post_body,
        grid=(ng, N // BM),
        in_specs=[
            pl.BlockSpec((2, BM, D), lambda k, i: (0, k * 10 + i, 0)),
            pl.BlockSpec((BM, D), lambda k, i: (k * 10 + i, 0)),
            pl.BlockSpec((2, BM, 1), lambda k, i: (0, k * 10 + i, 0)),
            pl.BlockSpec((1, D), lambda k, i: (0, 0)),
        ],
        out_specs=[
            pl.BlockSpec((BM, D), lambda k, i: (k * 10 + i, 0)),
            pl.BlockSpec((1, 2, D), lambda k, i: (k, 0, 0)),
        ],
        out_shape=[
            jax.ShapeDtypeStruct((ng * N, D), jnp.float32),
            jax.ShapeDtypeStruct((ng, 2, D), jnp.float32),
        ],
    )(mp, g, degp, b_gcn)


def _emb_body(enc_ref, bnp_ref, gb_ref, a_ref, out_ref):
    mu = bnp_ref[0, 0, :]
    rstd = bnp_ref[0, 1, :]
    xn = (enc_ref[...] - mu) * rstd * gb_ref[0] + gb_ref[1]
    a = a_ref[0, 0]
    out_ref[...] = jnp.where(xn >= 0, xn, a * xn)


def _tc_emb(enc, bnp, gb, a):
    return pl.pallas_call(
        _emb_body,
        grid=(N // BM,),
        in_specs=[
            pl.BlockSpec((BM, D), lambda i: (i, 0)),
            pl.BlockSpec((1, 2, D), lambda i: (0, 0, 0)),
            pl.BlockSpec((2, D), lambda i: (0, 0)),
            pl.BlockSpec((1, 1), lambda i: (0, 0)),
        ],
        out_specs=pl.BlockSpec((BM, D), lambda i: (i, 0)),
        out_shape=jax.ShapeDtypeStruct((N, D), jnp.float32),
    )(enc, bnp, gb, a)


def _proj1_body(enc_ref, bnp_ref, gb_ref, a_ref, w1_ref, b1_ref, m_ref, st_ref):
    i = pl.program_id(1)
    mu = bnp_ref[0, 0, :]
    rstd = bnp_ref[0, 1, :]
    xn = (enc_ref[...] - mu) * rstd * gb_ref[0] + gb_ref[1]
    a = a_ref[0, 0]
    hn = jnp.where(xn >= 0, xn, a * xn)
    m = jnp.dot(hn, w1_ref[...], preferred_element_type=jnp.float32) + b1_ref[...]
    m_ref[...] = m

    @pl.when(i == 0)
    def _():
        st_ref[...] = jnp.zeros_like(st_ref)

    st_ref[0, 0, :] += jnp.sum(m, axis=0)
    st_ref[0, 1, :] += jnp.sum(m * m, axis=0)


def _tc_proj1(enc, bnp, gb, a, w1, b1):
    return pl.pallas_call(
        _proj1_body,
        grid=(2, N // BM),
        in_specs=[
            pl.BlockSpec((BM, D), lambda k, i: (k * 10 + i, 0)),
            pl.BlockSpec((1, 2, D), lambda k, i: (k, 0, 0)),
            pl.BlockSpec((2, D), lambda k, i: (0, 0)),
            pl.BlockSpec((1, 1), lambda k, i: (0, 0)),
            pl.BlockSpec((D, H), lambda k, i: (0, 0)),
            pl.BlockSpec((1, H), lambda k, i: (0, 0)),
        ],
        out_specs=[
            pl.BlockSpec((BM, H), lambda k, i: (k * 10 + i, 0)),
            pl.BlockSpec((1, 2, H), lambda k, i: (k, 0, 0)),
        ],
        out_shape=[
            jax.ShapeDtypeStruct((2 * N, H), jnp.float32),
            jax.ShapeDtypeStruct((2, 2, H), jnp.float32),
        ],
    )(enc, bnp, gb, a, w1, b1)


def _proj2_body(m_ref, bnp_ref, gb_ref, a_ref, w2_ref, b2_ref, z_ref):
    mu = bnp_ref[0, 0, :]
    rstd = bnp_ref[0, 1, :]
    xn = (m_ref[...] - mu) * rstd * gb_ref[0] + gb_ref[1]
    a = a_ref[0, 0]
    hn = jnp.where(xn >= 0, xn, a * xn)
    h = jnp.dot(hn, w2_ref[...], preferred_element_type=jnp.float32) + b2_ref[...]
    nrm = jnp.maximum(jnp.sqrt(jnp.sum(h * h, axis=1, keepdims=True)), 1e-12)
    z_ref[...] = h / nrm


def _tc_proj2(m, bnp, gb, a, w2, b2):
    return pl.pallas_call(
        _proj2_body,
        grid=(2, N // BM),
        in_specs=[
            pl.BlockSpec((BM, H), lambda k, i: (k * 10 + i, 0)),
            pl.BlockSpec((1, 2, H), lambda k, i: (k, 0, 0)),
            pl.BlockSpec((2, H), lambda k, i: (0, 0)),
            pl.BlockSpec((1, 1), lambda k, i: (0, 0)),
            pl.BlockSpec((H, D), lambda k, i: (0, 0)),
            pl.BlockSpec((1, D), lambda k, i: (0, 0)),
        ],
        out_specs=pl.BlockSpec((BM, D), lambda k, i: (k * 10 + i, 0)),
        out_shape=jax.ShapeDtypeStruct((2 * N, D), jnp.float32),
    )(m, bnp, gb, a, w2, b2)


def _loss_body(z1i_ref, z2i_ref, z1j_ref, z2j_ref, out_ref,
               r1_s, r2_s, bb_s, cc_s, dd_s, r1d_s, r2d_s):
    i = pl.program_id(0)
    j = pl.program_id(1)

    @pl.when((i == 0) & (j == 0))
    def _():
        out_ref[...] = jnp.zeros_like(out_ref)

    @pl.when(j == 0)
    def _():
        r1_s[...] = jnp.zeros_like(r1_s)
        r2_s[...] = jnp.zeros_like(r2_s)
        bb_s[...] = jnp.zeros_like(bb_s)
        cc_s[...] = jnp.zeros_like(cc_s)

    z1i = z1i_ref[...]
    z2i = z2i_ref[...]
    z1j = z1j_ref[...]
    z2j = z2j_ref[...]
    dn = (((1,), (1,)), ((), ()))
    s11 = lax.dot_general(z1i, z1j, dn, preferred_element_type=jnp.float32)
    s22 = lax.dot_general(z2i, z2j, dn, preferred_element_type=jnp.float32)
    s12 = lax.dot_general(z1i, z2j, dn, preferred_element_type=jnp.float32)
    s21 = lax.dot_general(z2i, z1j, dn, preferred_element_type=jnp.float32)
    e11 = jnp.exp(s11 * INV_T)
    e22 = jnp.exp(s22 * INV_T)
    e12 = jnp.exp(s12 * INV_T)
    e21 = jnp.exp(s21 * INV_T)
    r1_s[...] += jnp.sum(e11, axis=1, keepdims=True)
    r2_s[...] += jnp.sum(e22, axis=1, keepdims=True)
    bb_s[...] += jnp.sum(e12, axis=1, keepdims=True)
    cc_s[...] += jnp.sum(e21, axis=1, keepdims=True)

    @pl.when(i == j)
    def _():
        ir = lax.broadcasted_iota(jnp.int32, (BM, BM), 0)
        ic = lax.broadcasted_iota(jnp.int32, (BM, BM), 1)
        dmask = ir == ic
        zero = jnp.zeros_like(s12)
        dd_s[...] = jnp.sum(jnp.where(dmask, s12, zero), axis=1, keepdims=True)
        r1d_s[...] = jnp.sum(jnp.where(dmask, e11, zero), axis=1, keepdims=True)
        r2d_s[...] = jnp.sum(jnp.where(dmask, e22, zero), axis=1, keepdims=True)

    @pl.when(j == (N // BM) - 1)
    def _():
        l1 = jnp.log(r1_s[...] + bb_s[...] - r1d_s[...]) - dd_s[...] * INV_T
        l2 = jnp.log(r2_s[...] + cc_s[...] - r2d_s[...]) - dd_s[...] * INV_T
        out_ref[...] += (0.5 / N) * jnp.sum(l1 + l2).reshape(1, 1)


def _tc_loss(z):
    nb = N // BM
    return pl.pallas_call(
        _loss_body,
        grid=(nb, nb),
        in_specs=[
            pl.BlockSpec((BM, D), lambda i, j: (i, 0)),
            pl.BlockSpec((BM, D), lambda i, j: (nb + i, 0)),
            pl.BlockSpec((BM, D), lambda i, j: (j, 0)),
            pl.BlockSpec((BM, D), lambda i, j: (nb + j, 0)),
        ],
        out_specs=pl.BlockSpec((1, 1), lambda i, j: (0, 0)),
        out_shape=jax.ShapeDtypeStruct((1, 1), jnp.float32),
        scratch_shapes=[pltpu.VMEM((BM, 1), jnp.float32) for _ in range(7)],
    )(z, z, z, z)


def kernel(x1, x2, x, edge_index1, edge_index2, edge_index, W_gcn, b_gcn,
           bn1_g, bn1_b, prelu1, W1, b1, bn2_g, bn2_b, prelu2, W2, b2):
    x3 = jnp.concatenate([x1, x2, x], axis=0)
    off = jnp.int32(N)
    ntile = _NC * _NS
    src3 = jnp.concatenate([edge_index1[0], edge_index2[0] + off,
                            edge_index[0] + 2 * off]
                           ).reshape(3 * ntile, TILE_E_MP // EBM, EBM)
    dst3 = jnp.concatenate([edge_index1[1], edge_index2[1],
                            edge_index[1]]
                           ).reshape(3 * ntile, TILE_E_MP // EBM, EBM)
    dstf3 = jnp.concatenate([edge_index1[1], edge_index2[1] + off,
                             edge_index[1] + 2 * off]
                            ).reshape(ntile, TILE_E_DEG // EB, EB)

    degp = _sc_degree(dstf3).reshape(2, DEG_PAD)[:, :NN].reshape(2, NN, 1)
    g = _tc_pre(x3, W_gcn, degp)
    zeros_chunk = jnp.zeros((WCH, D), jnp.float32)
    mp12 = _sc_message2(g, src3[:2 * ntile], dst3[:2 * ntile],
                        zeros_chunk)[:, :, :N, :].reshape(2, 2 * N, D)
    mp3 = _sc_message1(g, src3[2 * ntile:], dst3[2 * ntile:],
                       zeros_chunk)[:, :, :N, :].reshape(2, N, D)

    b2d = b_gcn.reshape(1, D)
    gb1 = jnp.stack([bn1_g, bn1_b], axis=0)
    a1 = prelu1.reshape(1, 1)

    enc12, st12 = _tc_post(mp12, g[:2 * N], degp[:, :2 * N], b2d, 2)
    mu1 = st12[:, 0, :] / N
    var1 = st12[:, 1, :] / N - mu1 * mu1
    bnp1 = jnp.stack([mu1, lax.rsqrt(var1 + BN_EPS)], axis=1)
    m, st2 = _tc_proj1(enc12, bnp1, gb1, a1, W1, b1.reshape(1, H))
    mu2 = st2[:, 0, :] / N
    var2 = st2[:, 1, :] / N - mu2 * mu2
    bnp2 = jnp.stack([mu2, lax.rsqrt(var2 + BN_EPS)], axis=1)
    gb2 = jnp.stack([bn2_g, bn2_b], axis=0)
    z = _tc_proj2(m, bnp2, gb2, prelu2.reshape(1, 1), W2, b2.reshape(1, D))
    loss = _tc_loss(z.astype(jnp.bfloat16)).reshape(())

    enc3, st3 = _tc_post(mp3, g[2 * N:], degp[:, 2 * N:], b2d, 1)
    mu3 = st3[:, 0, :] / N
    var3 = st3[:, 1, :] / N - mu3 * mu3
    bnp3 = jnp.stack([mu3, lax.rsqrt(var3 + BN_EPS)], axis=1)
    emb = _tc_emb(enc3, bnp3, gb1, a1)
    return (emb, loss)
